# baseline (speedup 1.0000x reference)
import jax
import jax.numpy as jnp
from jax import lax
from jax.experimental import pallas as pl
from jax.experimental.pallas import tpu as pltpu

K = 1024
D = 64
B = 16
HW = 1024
N = B * HW
COMMIT = 0.25

BPS = 4
STEPS = B // BPS


def _vq_body(z_ref, e_ref, esq_ref, q_ref, enc_ref, loss_ref,
             perp_ref, loss_acc, cnt_acc):
    g = pl.program_id(0)
    e = e_ref[...]
    em2 = e * -2.0
    esq = esq_ref[...]

    @pl.when(g == 0)
    def _init():
        loss_acc[0] = 0.0
        cnt_acc[...] = jnp.zeros_like(cnt_acc)

    kiota = lax.broadcasted_iota(jnp.int32, (HW, K), 1)
    for s in range(BPS):
        z = z_ref[s]
        zsq = jnp.sum(z * z, axis=0)[:, None]
        dotm2 = lax.dot_general(z, em2, (((0,), (1,)), ((), ())),
                                preferred_element_type=jnp.float32)
        dist = (zsq + esq) + dotm2

        idx = jnp.argmin(dist, axis=1)

        oh = (kiota == idx[:, None]).astype(jnp.float32)
        enc_ref[s] = oh

        qcm = lax.dot_general(e, oh, (((0,), (1,)), ((), ())),
                              preferred_element_type=jnp.float32)
        q_ref[s] = qcm

        loss_acc[0] += jnp.sum(jnp.min(dist, axis=1))
        cnt_acc[...] += jnp.sum(oh, axis=0)[None, :]

    @pl.when(g == STEPS - 1)
    def _fin():
        loss_ref[0] = (COMMIT / (N * D)) * loss_acc[0]
        p = cnt_acc[0, :] * (1.0 / N)
        perp_ref[0] = jnp.exp(-jnp.sum(p * jnp.log(p + 1e-10)))


def kernel(z_e, embedding):
    zb = z_e.reshape(B, D, HW)
    esq = jnp.sum(embedding ** 2, axis=1)[None, :]
    q, enc, loss, perp = pl.pallas_call(
        _vq_body,
        grid=(STEPS,),
        in_specs=[
            pl.BlockSpec((BPS, D, HW), lambda b: (b, 0, 0)),
            pl.BlockSpec((K, D), lambda b: (0, 0)),
            pl.BlockSpec((1, K), lambda b: (0, 0)),
        ],
        out_specs=[
            pl.BlockSpec((BPS, D, HW), lambda b: (b, 0, 0)),
            pl.BlockSpec((BPS, HW, K), lambda b: (b, 0, 0)),
            pl.BlockSpec(memory_space=pltpu.SMEM),
            pl.BlockSpec(memory_space=pltpu.SMEM),
        ],
        out_shape=[
            jax.ShapeDtypeStruct((B, D, HW), jnp.float32),
            jax.ShapeDtypeStruct((B, HW, K), jnp.float32),
            jax.ShapeDtypeStruct((1,), jnp.float32),
            jax.ShapeDtypeStruct((1,), jnp.float32),
        ],
        scratch_shapes=[
            pltpu.SMEM((1,), jnp.float32),
            pltpu.VMEM((1, K), jnp.float32),
        ],
    )(zb, embedding, esq)
    return (q.reshape(z_e.shape), loss[0], perp[0], enc.reshape(N, K))
